# Initial kernel scaffold; baseline (speedup 1.0000x reference)
import jax
import jax.numpy as jnp
from jax.experimental import pallas as pl


def kernel(x_real, x_imag, params):
    raise NotImplementedError("write your pallas kernel here")


# speedup vs baseline: 2.2566x; 2.2566x over previous
import jax
import jax.numpy as jnp
from jax.experimental import pallas as pl
from jax.experimental.pallas import tpu as pltpu

S, D, H, E, FF = 2048, 768, 12, 8, 3072
HD = D // H
RB = 256
T = 256
NT = S // T + E - 1
NPOS = NT * T
FS = 2
FF2 = FF // FS
GB = 8
PREC = jax.lax.Precision.DEFAULT

_ARB = lambda n: pltpu.CompilerParams(dimension_semantics=("arbitrary",) * n)


def _ln1_qkv_kernel(xr_ref, xi_ref, g_ref, wr_ref, wi_ref, br_ref, bi_ref,
                    or_ref, oi_ref):
    xr = xr_ref[...]
    xi = xi_ref[...]
    amp = jnp.sqrt(xr * xr + xi * xi + 1e-6)
    mean_amp = jnp.mean(amp, axis=-1, keepdims=True)
    s = (g_ref[...] * (amp / (mean_amp + 1e-6))) / (amp + 1e-6)
    hr = xr * s
    hi = xi * s
    wr = wr_ref[...]
    wi = wi_ref[...]
    or_ref[...] = (jnp.dot(hr, wr, precision=PREC)
                   - jnp.dot(hi, wi, precision=PREC) + br_ref[...])
    oi_ref[...] = (jnp.dot(hr, wi, precision=PREC)
                   + jnp.dot(hi, wr, precision=PREC) + bi_ref[...])


def _attn_kernel(qr_ref, qi_ref, kr_ref, ki_ref, vr_ref, vi_ref,
                 or_ref, oi_ref):
    dn = (((1,), (1,)), ((), ()))
    outs_r = []
    outs_i = []
    for u in range(2):
        sl = slice(u * HD, (u + 1) * HD)
        qr = qr_ref[:, sl]
        qi = qi_ref[:, sl]
        kr = kr_ref[:, sl]
        ki = ki_ref[:, sl]
        s = jax.lax.dot_general(qr, kr, dn, precision=PREC)
        s = s + jax.lax.dot_general(qi, ki, dn, precision=PREC)
        s = s * 0.125
        m = jnp.max(s, axis=-1, keepdims=True)
        ex = jnp.exp(s - m)
        a = ex / jnp.sum(ex, axis=-1, keepdims=True)
        outs_r.append(jnp.dot(a, vr_ref[:, sl], precision=PREC))
        outs_i.append(jnp.dot(a, vi_ref[:, sl], precision=PREC))
    or_ref[...] = jnp.concatenate(outs_r, axis=1)
    oi_ref[...] = jnp.concatenate(outs_i, axis=1)


def _oproj_kernel(aor_ref, aoi_ref, wr_ref, wi_ref, br_ref, bi_ref,
                  xr_ref, xi_ref, g2_ref,
                  xr1_ref, xi1_ref, hr2_ref, hi2_ref, idx_ref):
    aor = aor_ref[...]
    aoi = aoi_ref[...]
    wr = wr_ref[...]
    wi = wi_ref[...]
    ar = (jnp.dot(aor, wr, precision=PREC)
          - jnp.dot(aoi, wi, precision=PREC) + br_ref[...])
    ai = (jnp.dot(aor, wi, precision=PREC)
          + jnp.dot(aoi, wr, precision=PREC) + bi_ref[...])
    xr1 = xr_ref[...] + ar
    xi1 = xi_ref[...] + ai
    xr1_ref[...] = xr1
    xi1_ref[...] = xi1
    amp = jnp.sqrt(xr1 * xr1 + xi1 * xi1 + 1e-6)
    mean_amp = jnp.mean(amp, axis=-1, keepdims=True)
    s = (g2_ref[...] * (amp / (mean_amp + 1e-6))) / (amp + 1e-6)
    hr2 = xr1 * s
    hi2 = xi1 * s
    hr2_ref[...] = hr2
    hi2_ref[...] = hi2
    ph = jnp.arctan2(hi2, hr2)
    mc = jnp.mean(jnp.cos(ph), axis=-1, keepdims=True)
    ms = jnp.mean(jnp.sin(ph), axis=-1, keepdims=True)
    tp = jnp.arctan2(ms, mc)
    npz = (tp + jnp.pi) / (2.0 * jnp.pi)
    idx = jnp.clip(jnp.floor(npz * E).astype(jnp.int32), 0, E - 1)
    idx_ref[...] = idx.reshape(1, RB, 1)


def _cumsum_lanes(m):
    r = m
    off = 1
    while off < S:
        shifted = jnp.concatenate(
            [jnp.zeros((1, off), m.dtype), r[:, :S - off]], axis=1)
        r = r + shifted
        off *= 2
    return r


def _plan_kernel(idx_ref, pos_ref, inv_ref, texp_ref, tvalid_ref):
    idx = idx_ref[...]
    tvec = jax.lax.broadcasted_iota(jnp.int32, (1, 128), 1)
    pos = jnp.zeros((1, S), jnp.int32)
    texp = jnp.zeros((1, 128), jnp.int32)
    base = jnp.zeros((), jnp.int32)
    run_tiles = jnp.zeros((), jnp.int32)
    for e in range(E):
        m = (idx == e).astype(jnp.int32)
        c = jnp.sum(m)
        r = _cumsum_lanes(m) - m
        pos = pos + m * (base + r)
        nt_e = (c + T - 1) // T
        base = base + nt_e * T
        run_tiles = run_tiles + nt_e
        texp = texp + (tvec >= run_tiles).astype(jnp.int32)
    pos_ref[...] = pos
    texp_ref[...] = jnp.minimum(texp, E - 1)
    tvalid_ref[...] = (tvec < run_tiles).astype(jnp.int32)
    tok = jax.lax.broadcasted_iota(jnp.int32, (1, S), 1)
    PB = 256
    for pb in range(NPOS // PB):
        pcol = jax.lax.broadcasted_iota(jnp.int32, (PB, 1), 0) + pb * PB
        eq = (pcol == pos).astype(jnp.int32) * tok
        inv_ref[pb * PB:(pb + 1) * PB, :] = jnp.sum(eq, axis=1, keepdims=True)


def _gather_kernel(idxs_ref, *refs):
    del idxs_ref
    rr = refs[:GB]
    ri = refs[GB:2 * GB]
    or_ref = refs[2 * GB]
    oi_ref = refs[2 * GB + 1]
    or_ref[...] = jnp.concatenate([r[...].reshape(1, D) for r in rr], axis=0)
    oi_ref[...] = jnp.concatenate([r[...].reshape(1, D) for r in ri], axis=0)


def _combine_kernel(pos_ref, *refs):
    del pos_ref
    rr = refs[:GB]
    ri = refs[GB:2 * GB]
    xr1_ref = refs[2 * GB]
    xi1_ref = refs[2 * GB + 1]
    or_ref = refs[2 * GB + 2]
    oi_ref = refs[2 * GB + 3]
    gr = jnp.concatenate([r[...].reshape(1, D) for r in rr], axis=0)
    gi = jnp.concatenate([r[...].reshape(1, D) for r in ri], axis=0)
    or_ref[...] = xr1_ref[...] + gr
    oi_ref[...] = xi1_ref[...] + gi


def _expert_kernel(texp_ref, tvalid_ref, xr_ref, xi_ref,
                   w1r_ref, w1i_ref, b1r_ref, b1i_ref, mb_ref,
                   w2r_ref, w2i_ref, b2r_ref, b2i_ref, er_ref, ei_ref):
    t = pl.program_id(0)
    f = pl.program_id(1)

    @pl.when(tvalid_ref[t] == 1)
    def _():
        xr = xr_ref[...]
        xi = xi_ref[...]
        w1r = w1r_ref[0]
        w1i = w1i_ref[0]
        hr = (jnp.dot(xr, w1r, precision=PREC)
              - jnp.dot(xi, w1i, precision=PREC) + b1r_ref[0])
        hi = (jnp.dot(xr, w1i, precision=PREC)
              + jnp.dot(xi, w1r, precision=PREC) + b1i_ref[0])
        amp = jnp.sqrt(hr * hr + hi * hi + 1e-10)
        oa = jnp.maximum(amp + mb_ref[0], 0.0)
        sc = oa / (amp + 1e-10)
        hr = hr * sc
        hi = hi * sc
        w2r = w2r_ref[0]
        w2i = w2i_ref[0]
        er = (jnp.dot(hr, w2r, precision=PREC)
              - jnp.dot(hi, w2i, precision=PREC))
        ei = (jnp.dot(hr, w2i, precision=PREC)
              + jnp.dot(hi, w2r, precision=PREC))

        @pl.when(f == 0)
        def _():
            er_ref[...] = er + b2r_ref[0]
            ei_ref[...] = ei + b2i_ref[0]

        @pl.when(f != 0)
        def _():
            er_ref[...] += er
            ei_ref[...] += ei


def _row_spec(j):
    return pl.BlockSpec((1, 1, D), lambda i, idxs: (idxs[i * GB + j], 0, 0))


def kernel(x_real, x_imag, params):
    p = params
    B = x_real.shape[0]
    xr2 = x_real.reshape(S, D)
    xi2 = x_imag.reshape(S, D)
    g1 = p["gamma1"].reshape(1, D)
    g2 = p["gamma2"].reshape(1, D)
    wr_all = jnp.concatenate([p["Wq_r"], p["Wk_r"], p["Wv_r"]], axis=1)
    wi_all = jnp.concatenate([p["Wq_i"], p["Wk_i"], p["Wv_i"]], axis=1)
    br_all = jnp.concatenate([p["bq_r"], p["bk_r"], p["bv_r"]]).reshape(1, 3 * D)
    bi_all = jnp.concatenate([p["bq_i"], p["bk_i"], p["bv_i"]]).reshape(1, 3 * D)

    f32 = jnp.float32
    qkvr, qkvi = pl.pallas_call(
        _ln1_qkv_kernel,
        grid=(S // RB,),
        in_specs=[
            pl.BlockSpec((RB, D), lambda i: (i, 0)),
            pl.BlockSpec((RB, D), lambda i: (i, 0)),
            pl.BlockSpec((1, D), lambda i: (0, 0)),
            pl.BlockSpec((D, 3 * D), lambda i: (0, 0)),
            pl.BlockSpec((D, 3 * D), lambda i: (0, 0)),
            pl.BlockSpec((1, 3 * D), lambda i: (0, 0)),
            pl.BlockSpec((1, 3 * D), lambda i: (0, 0)),
        ],
        out_specs=[pl.BlockSpec((RB, 3 * D), lambda i: (i, 0))] * 2,
        out_shape=[jax.ShapeDtypeStruct((S, 3 * D), f32)] * 2,
        compiler_params=_ARB(1),
    )(xr2, xi2, g1, wr_all, wi_all, br_all, bi_all)

    HP = H // 2
    aor, aoi = pl.pallas_call(
        _attn_kernel,
        grid=(HP, S // RB),
        in_specs=[
            pl.BlockSpec((RB, 2 * HD), lambda h, r: (r, h)),
            pl.BlockSpec((RB, 2 * HD), lambda h, r: (r, h)),
            pl.BlockSpec((S, 2 * HD), lambda h, r: (0, HP + h)),
            pl.BlockSpec((S, 2 * HD), lambda h, r: (0, HP + h)),
            pl.BlockSpec((S, 2 * HD), lambda h, r: (0, 2 * HP + h)),
            pl.BlockSpec((S, 2 * HD), lambda h, r: (0, 2 * HP + h)),
        ],
        out_specs=[pl.BlockSpec((RB, 2 * HD), lambda h, r: (r, h))] * 2,
        out_shape=[jax.ShapeDtypeStruct((S, D), f32)] * 2,
        compiler_params=_ARB(2),
    )(qkvr, qkvi, qkvr, qkvi, qkvr, qkvi)

    xr1, xi1, hr2, hi2, idx3 = pl.pallas_call(
        _oproj_kernel,
        grid=(S // RB,),
        in_specs=[
            pl.BlockSpec((RB, D), lambda i: (i, 0)),
            pl.BlockSpec((RB, D), lambda i: (i, 0)),
            pl.BlockSpec((D, D), lambda i: (0, 0)),
            pl.BlockSpec((D, D), lambda i: (0, 0)),
            pl.BlockSpec((1, D), lambda i: (0, 0)),
            pl.BlockSpec((1, D), lambda i: (0, 0)),
            pl.BlockSpec((RB, D), lambda i: (i, 0)),
            pl.BlockSpec((RB, D), lambda i: (i, 0)),
            pl.BlockSpec((1, D), lambda i: (0, 0)),
        ],
        out_specs=[
            pl.BlockSpec((RB, D), lambda i: (i, 0)),
            pl.BlockSpec((RB, D), lambda i: (i, 0)),
            pl.BlockSpec((RB, D), lambda i: (i, 0)),
            pl.BlockSpec((RB, D), lambda i: (i, 0)),
            pl.BlockSpec((1, RB, 1), lambda i: (i, 0, 0)),
        ],
        out_shape=[
            jax.ShapeDtypeStruct((S, D), f32),
            jax.ShapeDtypeStruct((S, D), f32),
            jax.ShapeDtypeStruct((S, D), f32),
            jax.ShapeDtypeStruct((S, D), f32),
            jax.ShapeDtypeStruct((S // RB, RB, 1), jnp.int32),
        ],
        compiler_params=_ARB(1),
    )(aor, aoi, p["Wo_r"], p["Wo_i"], p["bo_r"].reshape(1, D),
      p["bo_i"].reshape(1, D), xr2, xi2, g2)

    idx_row = idx3.reshape(1, S)
    pos2, inv2, texp2, tvalid2 = pl.pallas_call(
        _plan_kernel,
        grid=(1,),
        in_specs=[pl.BlockSpec((1, S), lambda i: (0, 0))],
        out_specs=[
            pl.BlockSpec((1, S), lambda i: (0, 0)),
            pl.BlockSpec((NPOS, 1), lambda i: (0, 0)),
            pl.BlockSpec((1, 128), lambda i: (0, 0)),
            pl.BlockSpec((1, 128), lambda i: (0, 0)),
        ],
        out_shape=[
            jax.ShapeDtypeStruct((1, S), jnp.int32),
            jax.ShapeDtypeStruct((NPOS, 1), jnp.int32),
            jax.ShapeDtypeStruct((1, 128), jnp.int32),
            jax.ShapeDtypeStruct((1, 128), jnp.int32),
        ],
        compiler_params=_ARB(1),
    )(idx_row)
    pos = pos2.reshape(S)
    inv = inv2.reshape(NPOS)
    texp = texp2[0, :NT]
    tvalid = tvalid2[0, :NT]

    hr2r = hr2.reshape(S, 1, D)
    hi2r = hi2.reshape(S, 1, D)
    bufr, bufi = pl.pallas_call(
        _gather_kernel,
        grid_spec=pltpu.PrefetchScalarGridSpec(
            num_scalar_prefetch=1,
            grid=(NPOS // GB,),
            in_specs=[_row_spec(j) for j in range(GB)] * 2,
            out_specs=[pl.BlockSpec((GB, D), lambda i, idxs: (i, 0))] * 2,
        ),
        out_shape=[jax.ShapeDtypeStruct((NPOS, D), f32)] * 2,
        compiler_params=_ARB(1),
    )(inv, *([hr2r] * GB), *([hi2r] * GB))

    def _feff(t, f):
        return jax.lax.select((t % 2) == 0, f, FS - 1 - f)

    er, ei = pl.pallas_call(
        _expert_kernel,
        grid_spec=pltpu.PrefetchScalarGridSpec(
            num_scalar_prefetch=2,
            grid=(NT, FS),
            in_specs=[
                pl.BlockSpec((T, D), lambda t, f, te, tv: (t, 0)),
                pl.BlockSpec((T, D), lambda t, f, te, tv: (t, 0)),
                pl.BlockSpec((1, D, FF2), lambda t, f, te, tv: (te[t], 0, _feff(t, f))),
                pl.BlockSpec((1, D, FF2), lambda t, f, te, tv: (te[t], 0, _feff(t, f))),
                pl.BlockSpec((1, 1, FF2), lambda t, f, te, tv: (te[t], 0, _feff(t, f))),
                pl.BlockSpec((1, 1, FF2), lambda t, f, te, tv: (te[t], 0, _feff(t, f))),
                pl.BlockSpec((1, 1, FF2), lambda t, f, te, tv: (te[t], 0, _feff(t, f))),
                pl.BlockSpec((1, FF2, D), lambda t, f, te, tv: (te[t], _feff(t, f), 0)),
                pl.BlockSpec((1, FF2, D), lambda t, f, te, tv: (te[t], _feff(t, f), 0)),
                pl.BlockSpec((1, 1, D), lambda t, f, te, tv: (te[t], 0, 0)),
                pl.BlockSpec((1, 1, D), lambda t, f, te, tv: (te[t], 0, 0)),
            ],
            out_specs=[pl.BlockSpec((T, D), lambda t, f, te, tv: (t, 0))] * 2,
        ),
        out_shape=[jax.ShapeDtypeStruct((NPOS, D), f32)] * 2,
        compiler_params=_ARB(2),
    )(texp, tvalid, bufr, bufi,
      p["W1_r"], p["W1_i"],
      p["b1_r"].reshape(E, 1, FF), p["b1_i"].reshape(E, 1, FF),
      p["mb"].reshape(E, 1, FF),
      p["W2_r"], p["W2_i"],
      p["b2_r"].reshape(E, 1, D), p["b2_i"].reshape(E, 1, D))

    outr, outi = pl.pallas_call(
        _combine_kernel,
        grid_spec=pltpu.PrefetchScalarGridSpec(
            num_scalar_prefetch=1,
            grid=(S // GB,),
            in_specs=([_row_spec(j) for j in range(GB)] * 2
                      + [pl.BlockSpec((GB, D), lambda i, idxs: (i, 0))] * 2),
            out_specs=[pl.BlockSpec((GB, D), lambda i, idxs: (i, 0))] * 2,
        ),
        out_shape=[jax.ShapeDtypeStruct((S, D), f32)] * 2,
        compiler_params=_ARB(1),
    )(pos, *([er.reshape(NPOS, 1, D)] * GB), *([ei.reshape(NPOS, 1, D)] * GB),
      xr1, xi1)

    return jnp.concatenate([outr, outi], axis=-1).reshape(B, S, 2 * D)


# speedup vs baseline: 3.0933x; 1.3708x over previous
import jax
import jax.numpy as jnp
from jax.experimental import pallas as pl
from jax.experimental.pallas import tpu as pltpu
from jax.experimental.pallas import tpu_sc as plsc

S, D, H, E, FF = 2048, 768, 12, 8, 3072
HD = D // H
RB = 256
T = 256
NT = S // T + E - 1
NPOS = NT * T
FS = 2
FF2 = FF // FS
GB = 8
PREC = jax.lax.Precision.DEFAULT

_ARB = lambda n: pltpu.CompilerParams(dimension_semantics=("arbitrary",) * n)

def _dtb(a, b):
    return jax.lax.dot(a.astype(jnp.bfloat16), b.astype(jnp.bfloat16),
                       preferred_element_type=jnp.float32)


def _dt(a, b):
    return jax.lax.dot(a.astype(jnp.bfloat16), b.astype(jnp.bfloat16),
                       preferred_element_type=jnp.float32)


def _dg(a, b, dn):
    return jax.lax.dot_general(a.astype(jnp.bfloat16), b.astype(jnp.bfloat16),
                               dn, preferred_element_type=jnp.float32)


def _ln1_qkv_kernel(xr_ref, xi_ref, g_ref, wr_ref, wi_ref, br_ref, bi_ref,
                    or_ref, oi_ref):
    xr = xr_ref[...]
    xi = xi_ref[...]
    amp = jnp.sqrt(xr * xr + xi * xi + 1e-6)
    mean_amp = jnp.mean(amp, axis=-1, keepdims=True)
    s = (g_ref[...] * (amp / (mean_amp + 1e-6))) / (amp + 1e-6)
    hr = xr * s
    hi = xi * s
    wr = wr_ref[...]
    wi = wi_ref[...]
    or_ref[...] = (_dt(hr, wr)
                   - _dt(hi, wi) + br_ref[...])
    oi_ref[...] = (_dt(hr, wi)
                   + _dt(hi, wr) + bi_ref[...])


def _attn_kernel(qr_ref, qi_ref, kr_ref, ki_ref, vr_ref, vi_ref,
                 or_ref, oi_ref):
    dn = (((1,), (1,)), ((), ()))
    outs_r = []
    outs_i = []
    for u in range(2):
        sl = slice(u * HD, (u + 1) * HD)
        qr = qr_ref[:, sl]
        qi = qi_ref[:, sl]
        kr = kr_ref[:, sl]
        ki = ki_ref[:, sl]
        s = _dg(qr, kr, dn)
        s = s + _dg(qi, ki, dn)
        s = s * 0.125
        m = jnp.max(s, axis=-1, keepdims=True)
        ex = jnp.exp(s - m)
        a = ex / jnp.sum(ex, axis=-1, keepdims=True)
        outs_r.append(_dt(a, vr_ref[:, sl]))
        outs_i.append(_dt(a, vi_ref[:, sl]))
    or_ref[...] = jnp.concatenate(outs_r, axis=1)
    oi_ref[...] = jnp.concatenate(outs_i, axis=1)


def _oproj_kernel(aor_ref, aoi_ref, wr_ref, wi_ref, br_ref, bi_ref,
                  xr_ref, xi_ref, g2_ref,
                  xr1_ref, xi1_ref, hr2_ref, hi2_ref, idx_ref):
    aor = aor_ref[...]
    aoi = aoi_ref[...]
    wr = wr_ref[...]
    wi = wi_ref[...]
    ar = (_dt(aor, wr)
          - _dt(aoi, wi) + br_ref[...])
    ai = (_dt(aor, wi)
          + _dt(aoi, wr) + bi_ref[...])
    xr1 = xr_ref[...] + ar
    xi1 = xi_ref[...] + ai
    xr1_ref[...] = xr1
    xi1_ref[...] = xi1
    amp = jnp.sqrt(xr1 * xr1 + xi1 * xi1 + 1e-6)
    mean_amp = jnp.mean(amp, axis=-1, keepdims=True)
    s = (g2_ref[...] * (amp / (mean_amp + 1e-6))) / (amp + 1e-6)
    hr2 = xr1 * s
    hi2 = xi1 * s
    hr2_ref[...] = hr2
    hi2_ref[...] = hi2
    pamp = jnp.sqrt(hr2 * hr2 + hi2 * hi2)
    mc = jnp.mean(jnp.where(pamp > 0.0, hr2 / pamp, 1.0),
                  axis=-1, keepdims=True)
    ms = jnp.mean(jnp.where(pamp > 0.0, hi2 / pamp, 0.0),
                  axis=-1, keepdims=True)
    ams = jnp.abs(ms)
    amc = jnp.abs(mc)
    idx = jnp.where(
        ms < 0.0,
        jnp.where(mc < 0.0,
                  jnp.where(ams >= amc, 1, 0),
                  jnp.where(ams > amc, 2, 3)),
        jnp.where(mc > 0.0,
                  jnp.where(ams >= amc, 5, 4),
                  jnp.where(ams > amc, 6, 7)),
    ).astype(jnp.int32)
    idx_ref[...] = idx.reshape(1, RB, 1)


def _cumsum_lanes(m):
    r = m
    off = 1
    while off < S:
        shifted = jnp.concatenate(
            [jnp.zeros((1, off), m.dtype), r[:, :S - off]], axis=1)
        r = r + shifted
        off *= 2
    return r


def _plan_kernel(idx_ref, pos_ref, inv_ref, texp_ref, tvalid_ref):
    idx = idx_ref[...]
    tvec = jax.lax.broadcasted_iota(jnp.int32, (1, 128), 1)
    pos = jnp.zeros((1, S), jnp.int32)
    texp = jnp.zeros((1, 128), jnp.int32)
    base = jnp.zeros((), jnp.int32)
    run_tiles = jnp.zeros((), jnp.int32)
    for e in range(E):
        m = (idx == e).astype(jnp.int32)
        c = jnp.sum(m)
        r = _cumsum_lanes(m) - m
        pos = pos + m * (base + r)
        nt_e = (c + T - 1) // T
        base = base + nt_e * T
        run_tiles = run_tiles + nt_e
        texp = texp + (tvec >= run_tiles).astype(jnp.int32)
    pos_ref[...] = pos
    texp_ref[...] = jnp.minimum(texp, E - 1)
    tvalid_ref[...] = (tvec < run_tiles).astype(jnp.int32)
    tok = jax.lax.broadcasted_iota(jnp.int32, (1, S), 1)
    PB = 256
    for pb in range(NPOS // PB):
        pcol = jax.lax.broadcasted_iota(jnp.int32, (PB, 1), 0) + pb * PB
        eq = (pcol == pos).astype(jnp.int32) * tok
        inv_ref[pb * PB:(pb + 1) * PB, :] = jnp.sum(eq, axis=1, keepdims=True)


def _sc_gather_pair(tr, ti, idx, n_out):
    chunk = n_out // 32
    mesh = plsc.VectorSubcoreMesh(core_axis_name="core",
                                  subcore_axis_name="subcore")

    @pl.kernel(
        out_type=[jax.ShapeDtypeStruct((n_out, D), jnp.float32)] * 2,
        mesh=mesh,
        scratch_types=[
            pltpu.VMEM((chunk,), jnp.int32),
            pltpu.VMEM((chunk, D), jnp.float32),
            pltpu.SemaphoreType.DMA,
        ])
    def k(tr_hbm, ti_hbm, i_hbm, or_hbm, oi_hbm, idx_v, rows_v, sem):
        wid = jax.lax.axis_index("subcore") * 2 + jax.lax.axis_index("core")
        base = wid * chunk
        pltpu.sync_copy(i_hbm.at[pl.ds(base, chunk)], idx_v)
        pltpu.async_copy(tr_hbm.at[idx_v], rows_v, sem).wait()
        pltpu.sync_copy(rows_v, or_hbm.at[pl.ds(base, chunk)])
        pltpu.async_copy(ti_hbm.at[idx_v], rows_v, sem).wait()
        pltpu.sync_copy(rows_v, oi_hbm.at[pl.ds(base, chunk)])

    return k(tr, ti, idx)


def _final_kernel(xr1_ref, xi1_ref, gr_ref, gi_ref, out_ref):
    out_ref[:, :D] = xr1_ref[...] + gr_ref[...]
    out_ref[:, D:] = xi1_ref[...] + gi_ref[...]


def _expert_kernel(texp_ref, tvalid_ref, xr_ref, xi_ref,
                   w1r_ref, w1i_ref, b1r_ref, b1i_ref, mb_ref,
                   w2r_ref, w2i_ref, b2r_ref, b2i_ref, er_ref, ei_ref):
    t = pl.program_id(0)
    f = pl.program_id(1)

    @pl.when(tvalid_ref[t] == 1)
    def _():
        xr = xr_ref[...]
        xi = xi_ref[...]
        w1r = w1r_ref[0]
        w1i = w1i_ref[0]
        hr = (_dtb(xr, w1r)
              - _dtb(xi, w1i) + b1r_ref[0])
        hi = (_dtb(xr, w1i)
              + _dtb(xi, w1r) + b1i_ref[0])
        amp = jnp.sqrt(hr * hr + hi * hi + 1e-10)
        oa = jnp.maximum(amp + mb_ref[0], 0.0)
        sc = oa / (amp + 1e-10)
        hr = hr * sc
        hi = hi * sc
        w2r = w2r_ref[0]
        w2i = w2i_ref[0]
        er = (_dtb(hr, w2r)
              - _dtb(hi, w2i))
        ei = (_dtb(hr, w2i)
              + _dtb(hi, w2r))

        @pl.when(f == 0)
        def _():
            er_ref[...] = er + b2r_ref[0]
            ei_ref[...] = ei + b2i_ref[0]

        @pl.when(f != 0)
        def _():
            er_ref[...] += er
            ei_ref[...] += ei


def kernel(x_real, x_imag, params):
    p = params
    B = x_real.shape[0]
    xr2 = x_real.reshape(S, D)
    xi2 = x_imag.reshape(S, D)
    g1 = p["gamma1"].reshape(1, D)
    g2 = p["gamma2"].reshape(1, D)
    wr_all = jnp.concatenate([p["Wq_r"], p["Wk_r"], p["Wv_r"]], axis=1)
    wi_all = jnp.concatenate([p["Wq_i"], p["Wk_i"], p["Wv_i"]], axis=1)
    br_all = jnp.concatenate([p["bq_r"], p["bk_r"], p["bv_r"]]).reshape(1, 3 * D)
    bi_all = jnp.concatenate([p["bq_i"], p["bk_i"], p["bv_i"]]).reshape(1, 3 * D)

    f32 = jnp.float32
    qkvr, qkvi = pl.pallas_call(
        _ln1_qkv_kernel,
        grid=(S // RB,),
        in_specs=[
            pl.BlockSpec((RB, D), lambda i: (i, 0)),
            pl.BlockSpec((RB, D), lambda i: (i, 0)),
            pl.BlockSpec((1, D), lambda i: (0, 0)),
            pl.BlockSpec((D, 3 * D), lambda i: (0, 0)),
            pl.BlockSpec((D, 3 * D), lambda i: (0, 0)),
            pl.BlockSpec((1, 3 * D), lambda i: (0, 0)),
            pl.BlockSpec((1, 3 * D), lambda i: (0, 0)),
        ],
        out_specs=[pl.BlockSpec((RB, 3 * D), lambda i: (i, 0))] * 2,
        out_shape=[jax.ShapeDtypeStruct((S, 3 * D), f32)] * 2,
        compiler_params=_ARB(1),
    )(xr2, xi2, g1, wr_all, wi_all, br_all, bi_all)

    HP = H // 2
    aor, aoi = pl.pallas_call(
        _attn_kernel,
        grid=(HP, S // RB),
        in_specs=[
            pl.BlockSpec((RB, 2 * HD), lambda h, r: (r, h)),
            pl.BlockSpec((RB, 2 * HD), lambda h, r: (r, h)),
            pl.BlockSpec((S, 2 * HD), lambda h, r: (0, HP + h)),
            pl.BlockSpec((S, 2 * HD), lambda h, r: (0, HP + h)),
            pl.BlockSpec((S, 2 * HD), lambda h, r: (0, 2 * HP + h)),
            pl.BlockSpec((S, 2 * HD), lambda h, r: (0, 2 * HP + h)),
        ],
        out_specs=[pl.BlockSpec((RB, 2 * HD), lambda h, r: (r, h))] * 2,
        out_shape=[jax.ShapeDtypeStruct((S, D), f32)] * 2,
        compiler_params=_ARB(2),
    )(qkvr, qkvi, qkvr, qkvi, qkvr, qkvi)

    xr1, xi1, hr2, hi2, idx3 = pl.pallas_call(
        _oproj_kernel,
        grid=(S // RB,),
        in_specs=[
            pl.BlockSpec((RB, D), lambda i: (i, 0)),
            pl.BlockSpec((RB, D), lambda i: (i, 0)),
            pl.BlockSpec((D, D), lambda i: (0, 0)),
            pl.BlockSpec((D, D), lambda i: (0, 0)),
            pl.BlockSpec((1, D), lambda i: (0, 0)),
            pl.BlockSpec((1, D), lambda i: (0, 0)),
            pl.BlockSpec((RB, D), lambda i: (i, 0)),
            pl.BlockSpec((RB, D), lambda i: (i, 0)),
            pl.BlockSpec((1, D), lambda i: (0, 0)),
        ],
        out_specs=[
            pl.BlockSpec((RB, D), lambda i: (i, 0)),
            pl.BlockSpec((RB, D), lambda i: (i, 0)),
            pl.BlockSpec((RB, D), lambda i: (i, 0)),
            pl.BlockSpec((RB, D), lambda i: (i, 0)),
            pl.BlockSpec((1, RB, 1), lambda i: (i, 0, 0)),
        ],
        out_shape=[
            jax.ShapeDtypeStruct((S, D), f32),
            jax.ShapeDtypeStruct((S, D), f32),
            jax.ShapeDtypeStruct((S, D), f32),
            jax.ShapeDtypeStruct((S, D), f32),
            jax.ShapeDtypeStruct((S // RB, RB, 1), jnp.int32),
        ],
        compiler_params=_ARB(1),
    )(aor, aoi, p["Wo_r"], p["Wo_i"], p["bo_r"].reshape(1, D),
      p["bo_i"].reshape(1, D), xr2, xi2, g2)

    idx_row = idx3.reshape(1, S)
    pos2, inv2, texp2, tvalid2 = pl.pallas_call(
        _plan_kernel,
        grid=(1,),
        in_specs=[pl.BlockSpec((1, S), lambda i: (0, 0))],
        out_specs=[
            pl.BlockSpec((1, S), lambda i: (0, 0)),
            pl.BlockSpec((NPOS, 1), lambda i: (0, 0)),
            pl.BlockSpec((1, 128), lambda i: (0, 0)),
            pl.BlockSpec((1, 128), lambda i: (0, 0)),
        ],
        out_shape=[
            jax.ShapeDtypeStruct((1, S), jnp.int32),
            jax.ShapeDtypeStruct((NPOS, 1), jnp.int32),
            jax.ShapeDtypeStruct((1, 128), jnp.int32),
            jax.ShapeDtypeStruct((1, 128), jnp.int32),
        ],
        compiler_params=_ARB(1),
    )(idx_row)
    pos = pos2.reshape(S)
    inv = inv2.reshape(NPOS)
    texp = texp2[0, :NT]
    tvalid = tvalid2[0, :NT]

    bufr, bufi = _sc_gather_pair(hr2, hi2, inv, NPOS)

    def _feff(t, f):
        return jax.lax.select((t % 2) == 0, f, FS - 1 - f)

    er, ei = pl.pallas_call(
        _expert_kernel,
        grid_spec=pltpu.PrefetchScalarGridSpec(
            num_scalar_prefetch=2,
            grid=(NT, FS),
            in_specs=[
                pl.BlockSpec((T, D), lambda t, f, te, tv: (t, 0)),
                pl.BlockSpec((T, D), lambda t, f, te, tv: (t, 0)),
                pl.BlockSpec((1, D, FF2), lambda t, f, te, tv: (te[t], 0, _feff(t, f))),
                pl.BlockSpec((1, D, FF2), lambda t, f, te, tv: (te[t], 0, _feff(t, f))),
                pl.BlockSpec((1, 1, FF2), lambda t, f, te, tv: (te[t], 0, _feff(t, f))),
                pl.BlockSpec((1, 1, FF2), lambda t, f, te, tv: (te[t], 0, _feff(t, f))),
                pl.BlockSpec((1, 1, FF2), lambda t, f, te, tv: (te[t], 0, _feff(t, f))),
                pl.BlockSpec((1, FF2, D), lambda t, f, te, tv: (te[t], _feff(t, f), 0)),
                pl.BlockSpec((1, FF2, D), lambda t, f, te, tv: (te[t], _feff(t, f), 0)),
                pl.BlockSpec((1, 1, D), lambda t, f, te, tv: (te[t], 0, 0)),
                pl.BlockSpec((1, 1, D), lambda t, f, te, tv: (te[t], 0, 0)),
            ],
            out_specs=[pl.BlockSpec((T, D), lambda t, f, te, tv: (t, 0))] * 2,
        ),
        out_shape=[jax.ShapeDtypeStruct((NPOS, D), f32)] * 2,
        compiler_params=_ARB(2),
    )(texp, tvalid, bufr, bufi,
      p["W1_r"], p["W1_i"],
      p["b1_r"].reshape(E, 1, FF), p["b1_i"].reshape(E, 1, FF),
      p["mb"].reshape(E, 1, FF),
      p["W2_r"], p["W2_i"],
      p["b2_r"].reshape(E, 1, D), p["b2_i"].reshape(E, 1, D))

    gr, gi = _sc_gather_pair(er, ei, pos, S)

    out = pl.pallas_call(
        _final_kernel,
        grid=(S // RB,),
        in_specs=[pl.BlockSpec((RB, D), lambda i: (i, 0))] * 4,
        out_specs=pl.BlockSpec((RB, 2 * D), lambda i: (i, 0)),
        out_shape=jax.ShapeDtypeStruct((S, 2 * D), f32),
        compiler_params=_ARB(1),
    )(xr1, xi1, gr, gi)

    return out.reshape(B, S, 2 * D)
